# baseline (speedup 1.0000x reference)
import jax
import jax.numpy as jnp
from jax.experimental import pallas as pl
from jax.experimental.pallas import tpu as pltpu

_B, _N, _F_IN, _H, _F_OUT = 16, 1024, 128, 32, 256
_K = 256
_NCHUNK = _N // _K
_KH = _K * _H


def _body(x_ref, gcn_w_ref, gcn_b_ref, net_ref, fc1_ref, fc1_b_ref,
          out_ref, sup_ref, hbuf_ref):
    i = pl.program_id(0)

    @pl.when(i == 0)
    def _compute_support():
        for b in range(_B):
            sup_ref[b] = jnp.dot(x_ref[b], gcn_w_ref[...],
                                 preferred_element_type=jnp.float32
                                 ).astype(jnp.bfloat16)

    for b in range(_B):
        h_b = jnp.dot(net_ref[b].astype(jnp.bfloat16), sup_ref[b],
                      preferred_element_type=jnp.float32)
        r_b = jnp.maximum(h_b + gcn_b_ref[...], 0.0)
        r4 = r_b.reshape(_K // 4, 4, _H)
        hbuf_ref[b] = jnp.concatenate([r4[:, c, :] for c in range(4)],
                                      axis=1)

    acc = jnp.zeros((_B, _F_OUT), jnp.float32)
    for g in range(_K // 4):
        acc += jnp.dot(hbuf_ref[:, g, :],
                       fc1_ref[g * (4 * _H):(g + 1) * (4 * _H), :],
                       preferred_element_type=jnp.float32)

    @pl.when(i == 0)
    def _init_out():
        out_ref[...] = acc + fc1_b_ref[...]

    @pl.when(i > 0)
    def _acc_out():
        out_ref[...] += acc


def kernel(x, network, gcn_w, gcn_b, fc1_w, fc1_b):
    gcn_b2 = gcn_b.reshape(1, _H)
    fc1_b2 = fc1_b.reshape(1, _F_OUT)

    out = pl.pallas_call(
        _body,
        grid=(_NCHUNK,),
        in_specs=[
            pl.BlockSpec((_B, _N, _F_IN), lambda i: (0, 0, 0)),
            pl.BlockSpec((_F_IN, _H), lambda i: (0, 0)),
            pl.BlockSpec((1, _H), lambda i: (0, 0)),
            pl.BlockSpec((_B, _K, _N), lambda i: (0, i, 0)),
            pl.BlockSpec((_KH, _F_OUT), lambda i: (i, 0)),
            pl.BlockSpec((1, _F_OUT), lambda i: (0, 0)),
        ],
        out_specs=pl.BlockSpec((_B, _F_OUT), lambda i: (0, 0)),
        out_shape=jax.ShapeDtypeStruct((_B, _F_OUT), jnp.float32),
        scratch_shapes=[
            pltpu.VMEM((_B, _N, _H), jnp.bfloat16),
            pltpu.VMEM((_B, _K // 4, 4 * _H), jnp.float32),
        ],
        compiler_params=pltpu.CompilerParams(
            dimension_semantics=("arbitrary",),
            vmem_limit_bytes=100 * 1024 * 1024,
        ),
    )(x, gcn_w, gcn_b2, network, fc1_w, fc1_b2)
    return out


# speedup vs baseline: 1.0326x; 1.0326x over previous
import jax
import jax.numpy as jnp
from jax.experimental import pallas as pl
from jax.experimental.pallas import tpu as pltpu

_B, _N, _F_IN, _H, _F_OUT = 16, 1024, 128, 32, 256
_K = 128
_NCHUNK = _N // _K
_KH = _K * _H


def _body(x_ref, gcn_w_ref, gcn_b_ref, net_ref, fc1_ref, fc1_b_ref,
          out_ref, sup_ref, hbuf_ref):
    i = pl.program_id(0)

    @pl.when(i == 0)
    def _compute_support():
        for b in range(_B):
            sup_ref[b] = jnp.dot(x_ref[b], gcn_w_ref[...],
                                 preferred_element_type=jnp.float32
                                 ).astype(jnp.bfloat16)

    for b in range(_B):
        h_b = jnp.dot(net_ref[b].astype(jnp.bfloat16), sup_ref[b],
                      preferred_element_type=jnp.float32)
        r_b = jnp.maximum(h_b + gcn_b_ref[...], 0.0)
        r4 = r_b.reshape(_K // 4, 4, _H)
        hbuf_ref[b] = jnp.concatenate([r4[:, c, :] for c in range(4)],
                                      axis=1)

    acc = jnp.zeros((_B, _F_OUT), jnp.float32)
    for g in range(_K // 4):
        acc += jnp.dot(hbuf_ref[:, g, :],
                       fc1_ref[g * (4 * _H):(g + 1) * (4 * _H), :],
                       preferred_element_type=jnp.float32)

    @pl.when(i == 0)
    def _init_out():
        out_ref[...] = acc + fc1_b_ref[...]

    @pl.when(i > 0)
    def _acc_out():
        out_ref[...] += acc


def kernel(x, network, gcn_w, gcn_b, fc1_w, fc1_b):
    gcn_b2 = gcn_b.reshape(1, _H)
    fc1_b2 = fc1_b.reshape(1, _F_OUT)

    out = pl.pallas_call(
        _body,
        grid=(_NCHUNK,),
        in_specs=[
            pl.BlockSpec((_B, _N, _F_IN), lambda i: (0, 0, 0)),
            pl.BlockSpec((_F_IN, _H), lambda i: (0, 0)),
            pl.BlockSpec((1, _H), lambda i: (0, 0)),
            pl.BlockSpec((_B, _K, _N), lambda i: (0, i, 0)),
            pl.BlockSpec((_KH, _F_OUT), lambda i: (i, 0)),
            pl.BlockSpec((1, _F_OUT), lambda i: (0, 0)),
        ],
        out_specs=pl.BlockSpec((_B, _F_OUT), lambda i: (0, 0)),
        out_shape=jax.ShapeDtypeStruct((_B, _F_OUT), jnp.float32),
        scratch_shapes=[
            pltpu.VMEM((_B, _N, _H), jnp.bfloat16),
            pltpu.VMEM((_B, _K // 4, 4 * _H), jnp.float32),
        ],
        compiler_params=pltpu.CompilerParams(
            dimension_semantics=("arbitrary",),
        ),
    )(x, gcn_w, gcn_b2, network, fc1_w, fc1_b2)
    return out
